# Initial kernel scaffold; baseline (speedup 1.0000x reference)
import jax
import jax.numpy as jnp
from jax.experimental import pallas as pl


def kernel(hn, he, edge_vec, emb, norm, fc1_w1, fc1_w2, fc2_w1, fc2_w2, lin1_w, lin2_w, edge_index):
    raise NotImplementedError("write your pallas kernel here")


# speedup vs baseline: 2.5861x; 2.5861x over previous
import functools
import math

import jax
import jax.numpy as jnp
from jax import lax
from jax.experimental import pallas as pl
from jax.experimental.pallas import tpu as pltpu
from jax.experimental.pallas import tpu_sc as plsc

N_NODES = 10000
N_EDGES = 160000
C = 16

NC = 2
NS = 16
NW = NC * NS
EPW = N_EDGES // NW
CH = 128
NFULL = EPW // CH
TAIL = EPW - NFULL * CH
NPAD = 10240
RPS = NPAD // NS

_SC_MESH = plsc.VectorSubcoreMesh(core_axis_name="c", subcore_axis_name="s")


def _gather_body(hn_hbm, src_hbm, dst_hbm, hs_out, hd_out,
                 idx_s, idx_d, rows_s, rows_d, idx_st, idx_dt,
                 rows_st, rows_dt, sem):
    cid = lax.axis_index("c")
    sid = lax.axis_index("s")
    wid = sid * NC + cid
    base = wid * EPW

    def chunk(j, carry):
        off = base + j * CH
        pltpu.sync_copy(src_hbm.at[pl.ds(off, CH)], idx_s)
        pltpu.sync_copy(dst_hbm.at[pl.ds(off, CH)], idx_d)
        pltpu.async_copy(hn_hbm.at[idx_s], rows_s, sem).wait()
        pltpu.async_copy(hn_hbm.at[idx_d], rows_d, sem).wait()
        pltpu.sync_copy(rows_s, hs_out.at[pl.ds(off, CH)])
        pltpu.sync_copy(rows_d, hd_out.at[pl.ds(off, CH)])
        return carry

    lax.fori_loop(0, NFULL, chunk, 0)

    off = base + NFULL * CH
    pltpu.sync_copy(src_hbm.at[pl.ds(off, TAIL)], idx_st)
    pltpu.sync_copy(dst_hbm.at[pl.ds(off, TAIL)], idx_dt)
    pltpu.async_copy(hn_hbm.at[idx_st], rows_st, sem).wait()
    pltpu.async_copy(hn_hbm.at[idx_dt], rows_dt, sem).wait()
    pltpu.sync_copy(rows_st, hs_out.at[pl.ds(off, TAIL)])
    pltpu.sync_copy(rows_dt, hd_out.at[pl.ds(off, TAIL)])


_sc_gather = pl.kernel(
    _gather_body,
    out_type=(jax.ShapeDtypeStruct((N_EDGES, C), jnp.float32),
              jax.ShapeDtypeStruct((N_EDGES, C), jnp.float32)),
    mesh=_SC_MESH,
    scratch_types=[
        pltpu.VMEM((CH,), jnp.int32),
        pltpu.VMEM((CH,), jnp.int32),
        pltpu.VMEM((CH, C), jnp.float32),
        pltpu.VMEM((CH, C), jnp.float32),
        pltpu.VMEM((TAIL,), jnp.int32),
        pltpu.VMEM((TAIL,), jnp.int32),
        pltpu.VMEM((TAIL, C), jnp.float32),
        pltpu.VMEM((TAIL, C), jnp.float32),
        pltpu.SemaphoreType.DMA,
    ],
    compiler_params=pltpu.CompilerParams(use_tc_tiling_on_sc=False),
    name="sc_gather_hn",
)


def _scatter_body(vals_hbm, dsti_hbm, out_hbm, shared,
                  zbuf, idx_b, val_b, idx_t, val_t):
    cid = lax.axis_index("c")
    sid = lax.axis_index("s")
    wid = sid * NC + cid

    def zrow(i, carry):
        zbuf[i, :] = jnp.zeros((C,), jnp.float32)
        return carry

    lax.fori_loop(0, RPS, zrow, 0)
    pltpu.sync_copy(zbuf, shared.at[pl.ds(sid * RPS, RPS)])
    plsc.subcore_barrier()

    base = wid * EPW

    def chunk(j, carry):
        off = base + j * CH
        pltpu.sync_copy(dsti_hbm.at[pl.ds(off, CH)], idx_b)
        pltpu.sync_copy(vals_hbm.at[pl.ds(off, CH)], val_b)
        pltpu.sync_copy(val_b, shared.at[idx_b], add=True)
        return carry

    lax.fori_loop(0, NFULL, chunk, 0)

    off = base + NFULL * CH
    pltpu.sync_copy(dsti_hbm.at[pl.ds(off, TAIL)], idx_t)
    pltpu.sync_copy(vals_hbm.at[pl.ds(off, TAIL)], val_t)
    pltpu.sync_copy(val_t, shared.at[idx_t], add=True)

    plsc.subcore_barrier()
    pltpu.sync_copy(shared.at[pl.ds(sid * RPS, RPS)],
                    out_hbm.at[pl.ds(cid * NPAD + sid * RPS, RPS)])


_sc_scatter = pl.kernel(
    _scatter_body,
    out_type=jax.ShapeDtypeStruct((NC * NPAD, C), jnp.float32),
    mesh=_SC_MESH,
    scratch_types=[
        pltpu.VMEM_SHARED((NPAD, C), jnp.float32),
        pltpu.VMEM((RPS, C), jnp.float32),
        pltpu.VMEM((CH,), jnp.int32),
        pltpu.VMEM((CH, C), jnp.float32),
        pltpu.VMEM((TAIL,), jnp.int32),
        pltpu.VMEM((TAIL, C), jnp.float32),
    ],
    compiler_params=pltpu.CompilerParams(use_tc_tiling_on_sc=False),
    name="sc_scatter_nodeftr",
)


BE = 2000
_SQRT2 = math.sqrt(2.0)
_INV_SQRT48 = 1.0 / math.sqrt(48.0)


def _edge_body(he_ref, hs_ref, hd_ref, emb_ref, nrm_ref,
               w1a_ref, w2a_ref, m1_ref, m2_ref, rm_ref, sm_ref,
               henew_ref, hesc_ref):
    he_b = he_ref[...]
    emb_b = emb_ref[...]
    rm = rm_ref[...]
    sm = sm_ref[...]

    h1 = jnp.maximum(jnp.dot(emb_b, w1a_ref[...],
                             preferred_element_type=jnp.float32), 0.0) * _SQRT2
    h2 = jnp.maximum(jnp.dot(emb_b, w2a_ref[...],
                             preferred_element_type=jnp.float32), 0.0) * _SQRT2

    g1 = (jnp.dot(he_b, m1_ref[0:16, :], preferred_element_type=jnp.float32)
          + jnp.dot(hs_ref[...], m1_ref[16:32, :], preferred_element_type=jnp.float32)
          + jnp.dot(hd_ref[...], m1_ref[32:48, :], preferred_element_type=jnp.float32))
    e1 = jnp.dot(h1, rm, preferred_element_type=jnp.float32) * g1
    tmp = jnp.maximum(jnp.dot(e1, sm, preferred_element_type=jnp.float32)
                      * _INV_SQRT48, 0.0)

    g2 = jnp.dot(tmp, m2_ref[...], preferred_element_type=jnp.float32)
    e2 = jnp.dot(h2, rm, preferred_element_type=jnp.float32) * g2
    he_new = he_b + jnp.dot(e2, sm, preferred_element_type=jnp.float32) * 0.25

    henew_ref[...] = he_new
    hesc_ref[...] = he_new * nrm_ref[...]


def _edge_mlp(he, hs, hd, emb, nrm2, w1a, w2a, m1, m2, rm, sm):
    grid = (N_EDGES // BE,)
    blk = lambda r, c: pl.BlockSpec((r, c), lambda i: (i, 0))
    full = lambda r, c: pl.BlockSpec((r, c), lambda i: (0, 0))
    return pl.pallas_call(
        _edge_body,
        grid=grid,
        in_specs=[blk(BE, C), blk(BE, C), blk(BE, C), blk(BE, 10), blk(BE, 1),
                  full(10, C), full(10, C), full(48, 256), full(C, 256),
                  full(C, 256), full(256, C)],
        out_specs=[pl.BlockSpec((BE, C), lambda i: (i, 0)),
                   pl.BlockSpec((BE, C), lambda i: (i, 0))],
        out_shape=(jax.ShapeDtypeStruct((N_EDGES, C), jnp.float32),
                   jax.ShapeDtypeStruct((N_EDGES, C), jnp.float32)),
        name="tc_edge_mlp",
    )(he, hs, hd, emb, nrm2, w1a, w2a, m1, m2, rm, sm)


BN = 2000


def _node_body(hn_ref, nf0_ref, nf1_ref, l1a_ref, l1b_ref, l2_ref, out_ref):
    hn_b = hn_ref[...]
    nf = nf0_ref[...] + nf1_ref[...]
    h = jnp.maximum(jnp.dot(hn_b, l1a_ref[...], preferred_element_type=jnp.float32)
                    + jnp.dot(nf, l1b_ref[...], preferred_element_type=jnp.float32),
                    0.0) * _SQRT2
    out_ref[...] = hn_b + jnp.dot(h, l2_ref[...], preferred_element_type=jnp.float32)


def _node_update(hn, nf0, nf1, l1a, l1b, l2):
    grid = (N_NODES // BN,)
    blk = pl.BlockSpec((BN, C), lambda i: (i, 0))
    full = pl.BlockSpec((C, C), lambda i: (0, 0))
    return pl.pallas_call(
        _node_body,
        grid=grid,
        in_specs=[blk, blk, blk, full, full, full],
        out_specs=pl.BlockSpec((BN, C), lambda i: (i, 0)),
        out_shape=jax.ShapeDtypeStruct((N_NODES, C), jnp.float32),
        name="tc_node_update",
    )(hn, nf0, nf1, l1a, l1b, l2)


def kernel(hn, he, edge_vec, emb, norm, fc1_w1, fc1_w2, fc2_w1, fc2_w2,
           lin1_w, lin2_w, edge_index):
    del edge_vec
    src = edge_index[0]
    dst = edge_index[1]

    w1a = fc1_w1 * (1.0 / math.sqrt(10.0))
    w2a = fc2_w1 * (1.0 / math.sqrt(10.0))
    m1 = (fc1_w2 * 0.25).reshape(16, 48, 16).transpose(1, 0, 2).reshape(48, 256)
    m2 = (fc2_w2 * 0.25).reshape(16, 16, 16).transpose(1, 0, 2).reshape(16, 256)
    rm = jnp.kron(jnp.eye(16, dtype=jnp.float32), jnp.ones((1, 16), jnp.float32))
    sm = jnp.kron(jnp.ones((16, 1), jnp.float32), jnp.eye(16, dtype=jnp.float32))
    l1 = lin1_w * (1.0 / math.sqrt(32.0))
    l1a, l1b = l1[:C], l1[C:]
    l2 = lin2_w * 0.25

    hs, hd = _sc_gather(hn, src, dst)
    he_new, he_scaled = _edge_mlp(he, hs, hd, emb, norm[:, None],
                                  w1a, w2a, m1, m2, rm, sm)
    nf_parts = _sc_scatter(he_scaled, dst)
    nf = nf_parts.reshape(NC, NPAD, C)
    hn_new = _node_update(hn, nf[0, :N_NODES], nf[1, :N_NODES], l1a, l1b, l2)
    return hn_new, he_new


# speedup vs baseline: 2.6042x; 1.0070x over previous
import functools
import math

import jax
import jax.numpy as jnp
from jax import lax
from jax.experimental import pallas as pl
from jax.experimental.pallas import tpu as pltpu
from jax.experimental.pallas import tpu_sc as plsc

N_NODES = 10000
N_EDGES = 160000
C = 16

NC = 2
NS = 16
NW = NC * NS
EPW = N_EDGES // NW
GCH = 1000
GNC = EPW // GCH
SCH = 128
SNF = EPW // SCH
STL = EPW - SNF * SCH
NPAD = 10240
RPS = NPAD // NS

_SC_MESH = plsc.VectorSubcoreMesh(core_axis_name="c", subcore_axis_name="s")
_SC_PARAMS = pltpu.CompilerParams(use_tc_tiling_on_sc=False)


def _gather_body(hn_hbm, src_hbm, dst_hbm, hs_out, hd_out,
                 idx_s, idx_d, rows_s, rows_d, sem_i, sem_g, sem_o):
    cid = lax.axis_index("c")
    sid = lax.axis_index("s")
    wid = sid * NC + cid
    base = wid * EPW

    pltpu.async_copy(src_hbm.at[pl.ds(base, GCH)], idx_s.at[0], sem_i)
    pltpu.async_copy(dst_hbm.at[pl.ds(base, GCH)], idx_d.at[0], sem_i)

    def chunk(j, carry):
        p = lax.rem(j, 2)
        off = base + j * GCH
        pltpu.make_async_copy(src_hbm.at[pl.ds(off, GCH)], idx_s.at[p], sem_i).wait()
        pltpu.make_async_copy(dst_hbm.at[pl.ds(off, GCH)], idx_d.at[p], sem_i).wait()

        @pl.when(j + 1 < GNC)
        def _():
            off2 = off + GCH
            pltpu.async_copy(src_hbm.at[pl.ds(off2, GCH)], idx_s.at[1 - p], sem_i)
            pltpu.async_copy(dst_hbm.at[pl.ds(off2, GCH)], idx_d.at[1 - p], sem_i)

        @pl.when(j >= 2)
        def _():
            offw = off - 2 * GCH
            pltpu.make_async_copy(rows_s.at[p], hs_out.at[pl.ds(offw, GCH)],
                                  sem_o.at[p]).wait()
            pltpu.make_async_copy(rows_d.at[p], hd_out.at[pl.ds(offw, GCH)],
                                  sem_o.at[p]).wait()

        g1 = pltpu.async_copy(hn_hbm.at[idx_s.at[p]], rows_s.at[p], sem_g)
        g2 = pltpu.async_copy(hn_hbm.at[idx_d.at[p]], rows_d.at[p], sem_g)
        g1.wait()
        g2.wait()
        pltpu.async_copy(rows_s.at[p], hs_out.at[pl.ds(off, GCH)], sem_o.at[p])
        pltpu.async_copy(rows_d.at[p], hd_out.at[pl.ds(off, GCH)], sem_o.at[p])
        return carry

    lax.fori_loop(0, GNC, chunk, 0)

    for t in (GNC - 2, GNC - 1):
        p = t % 2
        off = base + t * GCH
        pltpu.make_async_copy(rows_s.at[p], hs_out.at[pl.ds(off, GCH)],
                              sem_o.at[p]).wait()
        pltpu.make_async_copy(rows_d.at[p], hd_out.at[pl.ds(off, GCH)],
                              sem_o.at[p]).wait()


_sc_gather = pl.kernel(
    _gather_body,
    out_type=(jax.ShapeDtypeStruct((N_EDGES, C), jnp.float32),
              jax.ShapeDtypeStruct((N_EDGES, C), jnp.float32)),
    mesh=_SC_MESH,
    scratch_types=[
        pltpu.VMEM((2, GCH), jnp.int32),
        pltpu.VMEM((2, GCH), jnp.int32),
        pltpu.VMEM((2, GCH, C), jnp.float32),
        pltpu.VMEM((2, GCH, C), jnp.float32),
        pltpu.SemaphoreType.DMA,
        pltpu.SemaphoreType.DMA,
        pltpu.SemaphoreType.DMA((2,)),
    ],
    compiler_params=_SC_PARAMS,
    name="sc_gather_hn",
)


def _scatter_body(vals_hbm, dsti_hbm, out_hbm, shared,
                  zbuf, idx_b, val_b, idx_t, val_t, sem_l, sem_s):
    cid = lax.axis_index("c")
    sid = lax.axis_index("s")
    wid = sid * NC + cid

    def zrow(i, carry):
        zbuf[i, :] = jnp.zeros((C,), jnp.float32)
        return carry

    lax.fori_loop(0, RPS, zrow, 0)
    pltpu.sync_copy(zbuf, shared.at[pl.ds(sid * RPS, RPS)])
    plsc.subcore_barrier()

    base = wid * EPW

    pltpu.async_copy(dsti_hbm.at[pl.ds(base, SCH)], idx_b.at[0], sem_l)
    pltpu.async_copy(vals_hbm.at[pl.ds(base, SCH)], val_b.at[0], sem_l)

    def chunk(j, carry):
        p = lax.rem(j, 3)
        p1 = lax.rem(j + 1, 3)
        off = base + j * SCH
        pltpu.make_async_copy(dsti_hbm.at[pl.ds(off, SCH)], idx_b.at[p], sem_l).wait()
        pltpu.make_async_copy(vals_hbm.at[pl.ds(off, SCH)], val_b.at[p], sem_l).wait()

        @pl.when(j >= 2)
        def _():
            pltpu.make_async_copy(val_b.at[p1], shared.at[idx_b.at[p1]],
                                  sem_s.at[p1]).wait()

        @pl.when(j + 1 < SNF)
        def _():
            off2 = off + SCH
            pltpu.async_copy(dsti_hbm.at[pl.ds(off2, SCH)], idx_b.at[p1], sem_l)
            pltpu.async_copy(vals_hbm.at[pl.ds(off2, SCH)], val_b.at[p1], sem_l)

        pltpu.async_copy(val_b.at[p], shared.at[idx_b.at[p]], sem_s.at[p], add=True)
        return carry

    lax.fori_loop(0, SNF, chunk, 0)

    for t in (SNF - 2, SNF - 1):
        p = t % 3
        pltpu.make_async_copy(val_b.at[p], shared.at[idx_b.at[p]], sem_s.at[p]).wait()

    off = base + SNF * SCH
    pltpu.sync_copy(dsti_hbm.at[pl.ds(off, STL)], idx_t)
    pltpu.sync_copy(vals_hbm.at[pl.ds(off, STL)], val_t)
    pltpu.sync_copy(val_t, shared.at[idx_t], add=True)

    plsc.subcore_barrier()
    pltpu.sync_copy(shared.at[pl.ds(sid * RPS, RPS)],
                    out_hbm.at[pl.ds(cid * NPAD + sid * RPS, RPS)])


_sc_scatter = pl.kernel(
    _scatter_body,
    out_type=jax.ShapeDtypeStruct((NC * NPAD, C), jnp.float32),
    mesh=_SC_MESH,
    scratch_types=[
        pltpu.VMEM_SHARED((NPAD, C), jnp.float32),
        pltpu.VMEM((RPS, C), jnp.float32),
        pltpu.VMEM((3, SCH), jnp.int32),
        pltpu.VMEM((3, SCH, C), jnp.float32),
        pltpu.VMEM((STL,), jnp.int32),
        pltpu.VMEM((STL, C), jnp.float32),
        pltpu.SemaphoreType.DMA,
        pltpu.SemaphoreType.DMA((3,)),
    ],
    compiler_params=_SC_PARAMS,
    name="sc_scatter_nodeftr",
)


BE = 2000
_SQRT2 = math.sqrt(2.0)
_INV_SQRT48 = 1.0 / math.sqrt(48.0)
_BF = jnp.bfloat16


def _bdot(a, b):
    return jax.lax.dot(a.astype(_BF), b, preferred_element_type=jnp.float32)


def _edge_body(he_ref, hs_ref, hd_ref, emb_ref, nrm_ref,
               w1a_ref, w2a_ref, m1_ref, m2_ref, rm_ref, sm_ref,
               henew_ref, hesc_ref):
    he_b = he_ref[...]
    emb_b = emb_ref[...]
    rm = rm_ref[...]
    sm = sm_ref[...]

    h1 = jnp.maximum(_bdot(emb_b, w1a_ref[...]), 0.0) * _SQRT2
    h2 = jnp.maximum(_bdot(emb_b, w2a_ref[...]), 0.0) * _SQRT2

    g1 = (_bdot(he_b, m1_ref[0:16, :])
          + _bdot(hs_ref[...], m1_ref[16:32, :])
          + _bdot(hd_ref[...], m1_ref[32:48, :]))
    e1 = _bdot(h1, rm) * g1
    tmp = jnp.maximum(_bdot(e1, sm) * _INV_SQRT48, 0.0)

    g2 = _bdot(tmp, m2_ref[...])
    e2 = _bdot(h2, rm) * g2
    he_new = he_b + _bdot(e2, sm) * 0.25

    henew_ref[...] = he_new
    hesc_ref[...] = he_new * nrm_ref[...]


def _edge_mlp(he, hs, hd, emb, nrm2, w1a, w2a, m1, m2, rm, sm):
    grid = (N_EDGES // BE,)
    blk = lambda r, c: pl.BlockSpec((r, c), lambda i: (i, 0))
    full = lambda r, c: pl.BlockSpec((r, c), lambda i: (0, 0))
    return pl.pallas_call(
        _edge_body,
        grid=grid,
        in_specs=[blk(BE, C), blk(BE, C), blk(BE, C), blk(BE, 10), blk(BE, 1),
                  full(10, C), full(10, C), full(48, 256), full(C, 256),
                  full(C, 256), full(256, C)],
        out_specs=[pl.BlockSpec((BE, C), lambda i: (i, 0)),
                   pl.BlockSpec((BE, C), lambda i: (i, 0))],
        out_shape=(jax.ShapeDtypeStruct((N_EDGES, C), jnp.float32),
                   jax.ShapeDtypeStruct((N_EDGES, C), jnp.float32)),
        name="tc_edge_mlp",
    )(he, hs, hd, emb, nrm2, w1a, w2a, m1, m2, rm, sm)


BN = 2000


def _node_body(hn_ref, nf0_ref, nf1_ref, l1a_ref, l1b_ref, l2_ref, out_ref):
    hn_b = hn_ref[...]
    nf = nf0_ref[...] + nf1_ref[...]
    h = jnp.maximum(jnp.dot(hn_b, l1a_ref[...], preferred_element_type=jnp.float32)
                    + jnp.dot(nf, l1b_ref[...], preferred_element_type=jnp.float32),
                    0.0) * _SQRT2
    out_ref[...] = hn_b + jnp.dot(h, l2_ref[...], preferred_element_type=jnp.float32)


def _node_update(hn, nf0, nf1, l1a, l1b, l2):
    grid = (N_NODES // BN,)
    blk = pl.BlockSpec((BN, C), lambda i: (i, 0))
    full = pl.BlockSpec((C, C), lambda i: (0, 0))
    return pl.pallas_call(
        _node_body,
        grid=grid,
        in_specs=[blk, blk, blk, full, full, full],
        out_specs=pl.BlockSpec((BN, C), lambda i: (i, 0)),
        out_shape=jax.ShapeDtypeStruct((N_NODES, C), jnp.float32),
        name="tc_node_update",
    )(hn, nf0, nf1, l1a, l1b, l2)


def kernel(hn, he, edge_vec, emb, norm, fc1_w1, fc1_w2, fc2_w1, fc2_w2,
           lin1_w, lin2_w, edge_index):
    del edge_vec
    src = edge_index[0]
    dst = edge_index[1]

    w1a = (fc1_w1 * (1.0 / math.sqrt(10.0))).astype(_BF)
    w2a = (fc2_w1 * (1.0 / math.sqrt(10.0))).astype(_BF)
    m1 = ((fc1_w2 * 0.25).reshape(16, 48, 16).transpose(1, 0, 2)
          .reshape(48, 256)).astype(_BF)
    m2 = ((fc2_w2 * 0.25).reshape(16, 16, 16).transpose(1, 0, 2)
          .reshape(16, 256)).astype(_BF)
    rm = jnp.kron(jnp.eye(16, dtype=_BF), jnp.ones((1, 16), _BF))
    sm = jnp.kron(jnp.ones((16, 1), _BF), jnp.eye(16, dtype=_BF))
    l1 = lin1_w * (1.0 / math.sqrt(32.0))
    l1a, l1b = l1[:C], l1[C:]
    l2 = lin2_w * 0.25

    hs, hd = _sc_gather(hn, src, dst)
    he_new, he_scaled = _edge_mlp(he, hs, hd, emb, norm[:, None],
                                  w1a, w2a, m1, m2, rm, sm)
    nf_parts = _sc_scatter(he_scaled, dst)
    nf = nf_parts.reshape(NC, NPAD, C)
    hn_new = _node_update(hn, nf[0, :N_NODES], nf[1, :N_NODES], l1a, l1b, l2)
    return hn_new, he_new


# speedup vs baseline: 4.8113x; 1.8475x over previous
import functools
import math

import jax
import jax.numpy as jnp
from jax import lax
from jax.experimental import pallas as pl
from jax.experimental.pallas import tpu as pltpu
from jax.experimental.pallas import tpu_sc as plsc

N_NODES = 10000
N_EDGES = 160000
C = 16

NC = 2
NS = 16
NW = NC * NS

EPW_LO = 5008
EPW_HI = 4992
GCH = 1248
NGRP = GCH // 16
GTL = 16

EPW = N_EDGES // NW
SCH = 128
SNF = EPW // SCH
STL = EPW - SNF * SCH
NPAD = 10240
RPS = NPAD // NS

_SC_MESH = plsc.VectorSubcoreMesh(core_axis_name="c", subcore_axis_name="s")
_SC_PARAMS = pltpu.CompilerParams(use_tc_tiling_on_sc=False,
                                  needs_layout_passes=False)


def _transpose_rows(rows, tb, ngrp):
    def grp(g, carry):
        rb = g * 16
        ridx = rb + lax.iota(jnp.int32, 16)
        for w in range(16):
            v = plsc.load_gather(rows, [ridx, jnp.full((16,), w, jnp.int32)])
            tb[w, pl.ds(rb, 16)] = v
        return carry
    lax.fori_loop(0, ngrp, grp, 0)


def _gather_body(hn_hbm, src_hbm, dst_hbm, hst_out, hdt_out,
                 idx_s, idx_d, rows_s, rows_d, tb_s, tb_d,
                 idx_st, idx_dt, rows_st, rows_dt, tb_st, tb_dt,
                 sem_g, sem_o):
    cid = lax.axis_index("c")
    sid = lax.axis_index("s")
    wid = sid * NC + cid
    lo = wid < 16
    base = jnp.where(lo, wid * EPW_LO, 16 * EPW_LO + (wid - 16) * EPW_HI)

    def chunk(j, carry):
        p = lax.rem(j, 2)
        off = base + j * GCH
        pltpu.sync_copy(src_hbm.at[pl.ds(off, GCH)], idx_s)
        pltpu.sync_copy(dst_hbm.at[pl.ds(off, GCH)], idx_d)
        g1 = pltpu.async_copy(hn_hbm.at[idx_s], rows_s, sem_g)
        g2 = pltpu.async_copy(hn_hbm.at[idx_d], rows_d, sem_g)
        g1.wait()
        g2.wait()

        @pl.when(j >= 2)
        def _():
            offw = off - 2 * GCH
            pltpu.make_async_copy(tb_s.at[p], hst_out.at[:, pl.ds(offw, GCH)],
                                  sem_o.at[p]).wait()
            pltpu.make_async_copy(tb_d.at[p], hdt_out.at[:, pl.ds(offw, GCH)],
                                  sem_o.at[p]).wait()

        _transpose_rows(rows_s, tb_s.at[p], NGRP)
        _transpose_rows(rows_d, tb_d.at[p], NGRP)
        pltpu.async_copy(tb_s.at[p], hst_out.at[:, pl.ds(off, GCH)], sem_o.at[p])
        pltpu.async_copy(tb_d.at[p], hdt_out.at[:, pl.ds(off, GCH)], sem_o.at[p])
        return carry

    lax.fori_loop(0, 4, chunk, 0)

    for t in (2, 3):
        p = t % 2
        off = base + t * GCH
        pltpu.make_async_copy(tb_s.at[p], hst_out.at[:, pl.ds(off, GCH)],
                              sem_o.at[p]).wait()
        pltpu.make_async_copy(tb_d.at[p], hdt_out.at[:, pl.ds(off, GCH)],
                              sem_o.at[p]).wait()

    @pl.when(lo)
    def _():
        off = base + 4 * GCH
        pltpu.sync_copy(src_hbm.at[pl.ds(off, GTL)], idx_st)
        pltpu.sync_copy(dst_hbm.at[pl.ds(off, GTL)], idx_dt)
        g1 = pltpu.async_copy(hn_hbm.at[idx_st], rows_st, sem_g)
        g2 = pltpu.async_copy(hn_hbm.at[idx_dt], rows_dt, sem_g)
        g1.wait()
        g2.wait()
        _transpose_rows(rows_st, tb_st, 1)
        _transpose_rows(rows_dt, tb_dt, 1)
        pltpu.sync_copy(tb_st, hst_out.at[:, pl.ds(off, GTL)])
        pltpu.sync_copy(tb_dt, hdt_out.at[:, pl.ds(off, GTL)])


_sc_gather = pl.kernel(
    _gather_body,
    out_type=(jax.ShapeDtypeStruct((C, N_EDGES), jnp.float32),
              jax.ShapeDtypeStruct((C, N_EDGES), jnp.float32)),
    mesh=_SC_MESH,
    scratch_types=[
        pltpu.VMEM((GCH,), jnp.int32),
        pltpu.VMEM((GCH,), jnp.int32),
        pltpu.VMEM((GCH, C), jnp.float32),
        pltpu.VMEM((GCH, C), jnp.float32),
        pltpu.VMEM((2, C, GCH), jnp.float32),
        pltpu.VMEM((2, C, GCH), jnp.float32),
        pltpu.VMEM((GTL,), jnp.int32),
        pltpu.VMEM((GTL,), jnp.int32),
        pltpu.VMEM((GTL, C), jnp.float32),
        pltpu.VMEM((GTL, C), jnp.float32),
        pltpu.VMEM((C, GTL), jnp.float32),
        pltpu.VMEM((C, GTL), jnp.float32),
        pltpu.SemaphoreType.DMA,
        pltpu.SemaphoreType.DMA((2,)),
    ],
    compiler_params=_SC_PARAMS,
    name="sc_gather_hn",
)


def _scatter_body(vals_hbm, dsti_hbm, out_hbm, shared,
                  zbuf, idx_b, val_b, idx_t, val_t, sem_l, sem_s):
    cid = lax.axis_index("c")
    sid = lax.axis_index("s")
    wid = sid * NC + cid

    def zrow(i, carry):
        zbuf[i, :] = jnp.zeros((C,), jnp.float32)
        return carry

    lax.fori_loop(0, RPS, zrow, 0)
    pltpu.sync_copy(zbuf, shared.at[pl.ds(sid * RPS, RPS)])
    plsc.subcore_barrier()

    base = wid * EPW

    pltpu.async_copy(dsti_hbm.at[pl.ds(base, SCH)], idx_b.at[0], sem_l)
    pltpu.async_copy(vals_hbm.at[pl.ds(base, SCH)], val_b.at[0], sem_l)

    def chunk(j, carry):
        p = lax.rem(j, 3)
        p1 = lax.rem(j + 1, 3)
        off = base + j * SCH
        pltpu.make_async_copy(dsti_hbm.at[pl.ds(off, SCH)], idx_b.at[p], sem_l).wait()
        pltpu.make_async_copy(vals_hbm.at[pl.ds(off, SCH)], val_b.at[p], sem_l).wait()

        @pl.when(j >= 2)
        def _():
            pltpu.make_async_copy(val_b.at[p1], shared.at[idx_b.at[p1]],
                                  sem_s.at[p1]).wait()

        @pl.when(j + 1 < SNF)
        def _():
            off2 = off + SCH
            pltpu.async_copy(dsti_hbm.at[pl.ds(off2, SCH)], idx_b.at[p1], sem_l)
            pltpu.async_copy(vals_hbm.at[pl.ds(off2, SCH)], val_b.at[p1], sem_l)

        pltpu.async_copy(val_b.at[p], shared.at[idx_b.at[p]], sem_s.at[p], add=True)
        return carry

    lax.fori_loop(0, SNF, chunk, 0)

    for t in (SNF - 2, SNF - 1):
        p = t % 3
        pltpu.make_async_copy(val_b.at[p], shared.at[idx_b.at[p]], sem_s.at[p]).wait()

    off = base + SNF * SCH
    pltpu.sync_copy(dsti_hbm.at[pl.ds(off, STL)], idx_t)
    pltpu.sync_copy(vals_hbm.at[pl.ds(off, STL)], val_t)
    pltpu.sync_copy(val_t, shared.at[idx_t], add=True)

    plsc.subcore_barrier()
    pltpu.sync_copy(shared.at[pl.ds(sid * RPS, RPS)],
                    out_hbm.at[pl.ds(cid * NPAD + sid * RPS, RPS)])


_sc_scatter = pl.kernel(
    _scatter_body,
    out_type=jax.ShapeDtypeStruct((NC * NPAD, C), jnp.float32),
    mesh=_SC_MESH,
    scratch_types=[
        pltpu.VMEM_SHARED((NPAD, C), jnp.float32),
        pltpu.VMEM((RPS, C), jnp.float32),
        pltpu.VMEM((3, SCH), jnp.int32),
        pltpu.VMEM((3, SCH, C), jnp.float32),
        pltpu.VMEM((STL,), jnp.int32),
        pltpu.VMEM((STL, C), jnp.float32),
        pltpu.SemaphoreType.DMA,
        pltpu.SemaphoreType.DMA((3,)),
    ],
    compiler_params=_SC_PARAMS,
    name="sc_scatter_nodeftr",
)


BT = 3200
_SQRT2 = math.sqrt(2.0)
_INV_SQRT48 = 1.0 / math.sqrt(48.0)
_BF = jnp.bfloat16


def _rep16(x):
    b = x.shape[1]
    return jnp.broadcast_to(x[:, None, :], (16, 16, b)).reshape(256, b)


def _sum16(x):
    b = x.shape[1]
    return x.reshape(16, 16, b).sum(axis=0)


def _edge_body(het_ref, hst_ref, hdt_ref, embt_ref, nrmt_ref,
               w12t_ref, m1t_ref, m2t_ref, henewt_ref, hesct_ref):
    het_b = het_ref[...]
    embt_b = embt_ref[...].astype(_BF)
    nrmt_b = nrmt_ref[...]

    h12 = jnp.maximum(jax.lax.dot(w12t_ref[...], embt_b,
                                  preferred_element_type=jnp.float32),
                      0.0) * _SQRT2
    h1t = h12[0:16, :]
    h2t = h12[16:32, :]

    featt = jnp.concatenate([het_b, hst_ref[...], hdt_ref[...]],
                            axis=0).astype(_BF)
    g1t = jax.lax.dot(m1t_ref[...], featt,
                      preferred_element_type=jnp.float32)
    tmpt = jnp.maximum(_sum16(_rep16(h1t) * g1t) * _INV_SQRT48, 0.0)

    g2t = jax.lax.dot(m2t_ref[...], tmpt.astype(_BF),
                      preferred_element_type=jnp.float32)
    he_newt = het_b + _sum16(_rep16(h2t) * g2t) * 0.25

    henewt_ref[...] = he_newt
    hesct_ref[...] = he_newt * nrmt_b


def _edge_mlp(het, hst, hdt, embt, nrmt, w12t, m1t, m2t):
    grid = (N_EDGES // BT,)
    col = lambda r: pl.BlockSpec((r, BT), lambda i: (0, i))
    full = lambda r, c: pl.BlockSpec((r, c), lambda i: (0, 0))
    return pl.pallas_call(
        _edge_body,
        grid=grid,
        in_specs=[col(C), col(C), col(C), col(10), col(1),
                  full(32, 10), full(256, 48), full(256, C)],
        out_specs=[col(C), col(C)],
        out_shape=(jax.ShapeDtypeStruct((C, N_EDGES), jnp.float32),
                   jax.ShapeDtypeStruct((C, N_EDGES), jnp.float32)),
        name="tc_edge_mlp",
    )(het, hst, hdt, embt, nrmt, w12t, m1t, m2t)


def _node_body(hnt_ref, nf0t_ref, nf1t_ref, l1t_ref, l2t_ref, out_ref):
    hnt_b = hnt_ref[...]
    nft = nf0t_ref[...] + nf1t_ref[...]
    cat = jnp.concatenate([hnt_b, nft], axis=0)
    h = jnp.maximum(jax.lax.dot(l1t_ref[...], cat,
                                preferred_element_type=jnp.float32),
                    0.0) * _SQRT2
    out_ref[...] = hnt_b + jax.lax.dot(l2t_ref[...], h,
                                       preferred_element_type=jnp.float32)


def _node_update(hnt, nf0t, nf1t, l1t, l2t):
    return pl.pallas_call(
        _node_body,
        out_shape=jax.ShapeDtypeStruct((C, N_NODES), jnp.float32),
        name="tc_node_update",
    )(hnt, nf0t, nf1t, l1t, l2t)


def kernel(hn, he, edge_vec, emb, norm, fc1_w1, fc1_w2, fc2_w1, fc2_w2,
           lin1_w, lin2_w, edge_index):
    del edge_vec
    src = edge_index[0]
    dst = edge_index[1]

    w12t = (jnp.concatenate([fc1_w1, fc2_w1], axis=1).T
            * (1.0 / math.sqrt(10.0))).astype(_BF)
    m1t = ((fc1_w2 * 0.25).reshape(16, 48, 16).transpose(1, 0, 2)
           .reshape(48, 256).T).astype(_BF)
    m2t = ((fc2_w2 * 0.25).reshape(16, 16, 16).transpose(1, 0, 2)
           .reshape(16, 256).T).astype(_BF)
    l1t = lin1_w.T * (1.0 / math.sqrt(32.0))
    l2t = lin2_w.T * 0.25

    hst, hdt = _sc_gather(hn, src, dst)
    he_newt, he_scaledt = _edge_mlp(he.T, hst, hdt, emb.T,
                                    norm.reshape(1, N_EDGES),
                                    w12t, m1t, m2t)
    nf_parts = _sc_scatter(he_scaledt.T, dst)
    nf = nf_parts.reshape(NC, NPAD, C)
    hn_newt = _node_update(hn.T, nf[0, :N_NODES].T, nf[1, :N_NODES].T,
                           l1t, l2t)
    return hn_newt.T, he_newt.T


# speedup vs baseline: 5.4763x; 1.1382x over previous
import functools
import math

import jax
import jax.numpy as jnp
from jax import lax
from jax.experimental import pallas as pl
from jax.experimental.pallas import tpu as pltpu
from jax.experimental.pallas import tpu_sc as plsc

N_NODES = 10000
N_EDGES = 160000
C = 16

NC = 2
NS = 16
NW = NC * NS

EPW_LO = 5008
EPW_HI = 4992
GCH = 1248
GNC = 4992 // GCH
NGRP = GCH // 16
GTL = 16

EPW = N_EDGES // NW
SCH = 128
SNF = EPW // SCH
STL = EPW - SNF * SCH
NPAD = 10240
RPS = NPAD // NS

_SC_MESH = plsc.VectorSubcoreMesh(core_axis_name="c", subcore_axis_name="s")
_SC_PARAMS = pltpu.CompilerParams(use_tc_tiling_on_sc=False,
                                  needs_layout_passes=False)


def _transpose_rows(rows, tb, ngrp):
    def grp(g, carry):
        rb = g * 16
        ridx = rb + lax.iota(jnp.int32, 16)
        for w in range(16):
            v = plsc.load_gather(rows, [ridx, jnp.full((16,), w, jnp.int32)])
            tb[w, pl.ds(rb, 16)] = v
        return carry
    lax.fori_loop(0, ngrp, grp, 0)


def _gather_body(hn_hbm, src_hbm, dst_hbm, hst_out, hdt_out,
                 idx_s, idx_d, rows_s, rows_d, tb_s, tb_d,
                 idx_st, idx_dt, rows_st, rows_dt, tb_st, tb_dt,
                 sem_g, sem_o):
    cid = lax.axis_index("c")
    sid = lax.axis_index("s")
    wid = sid * NC + cid
    lo = wid < 16
    base = jnp.where(lo, wid * EPW_LO, 16 * EPW_LO + (wid - 16) * EPW_HI)

    def chunk(j, carry):
        p = lax.rem(j, 2)
        off = base + j * GCH
        pltpu.sync_copy(src_hbm.at[pl.ds(off, GCH)], idx_s)
        pltpu.sync_copy(dst_hbm.at[pl.ds(off, GCH)], idx_d)
        g1 = pltpu.async_copy(hn_hbm.at[idx_s], rows_s, sem_g)
        g2 = pltpu.async_copy(hn_hbm.at[idx_d], rows_d, sem_g)
        g1.wait()
        g2.wait()

        @pl.when(j >= 2)
        def _():
            offw = off - 2 * GCH
            pltpu.make_async_copy(tb_s.at[p], hst_out.at[:, pl.ds(offw, GCH)],
                                  sem_o.at[p]).wait()
            pltpu.make_async_copy(tb_d.at[p], hdt_out.at[:, pl.ds(offw, GCH)],
                                  sem_o.at[p]).wait()

        _transpose_rows(rows_s, tb_s.at[p], NGRP)
        _transpose_rows(rows_d, tb_d.at[p], NGRP)
        pltpu.async_copy(tb_s.at[p], hst_out.at[:, pl.ds(off, GCH)], sem_o.at[p])
        pltpu.async_copy(tb_d.at[p], hdt_out.at[:, pl.ds(off, GCH)], sem_o.at[p])
        return carry

    lax.fori_loop(0, GNC, chunk, 0)

    for t in (GNC - 2, GNC - 1):
        p = t % 2
        off = base + t * GCH
        pltpu.make_async_copy(tb_s.at[p], hst_out.at[:, pl.ds(off, GCH)],
                              sem_o.at[p]).wait()
        pltpu.make_async_copy(tb_d.at[p], hdt_out.at[:, pl.ds(off, GCH)],
                              sem_o.at[p]).wait()

    @pl.when(lo)
    def _():
        off = base + GNC * GCH
        pltpu.sync_copy(src_hbm.at[pl.ds(off, GTL)], idx_st)
        pltpu.sync_copy(dst_hbm.at[pl.ds(off, GTL)], idx_dt)
        g1 = pltpu.async_copy(hn_hbm.at[idx_st], rows_st, sem_g)
        g2 = pltpu.async_copy(hn_hbm.at[idx_dt], rows_dt, sem_g)
        g1.wait()
        g2.wait()
        _transpose_rows(rows_st, tb_st, 1)
        _transpose_rows(rows_dt, tb_dt, 1)
        pltpu.sync_copy(tb_st, hst_out.at[:, pl.ds(off, GTL)])
        pltpu.sync_copy(tb_dt, hdt_out.at[:, pl.ds(off, GTL)])


_sc_gather = pl.kernel(
    _gather_body,
    out_type=(jax.ShapeDtypeStruct((C, N_EDGES), jnp.float32),
              jax.ShapeDtypeStruct((C, N_EDGES), jnp.float32)),
    mesh=_SC_MESH,
    scratch_types=[
        pltpu.VMEM((GCH,), jnp.int32),
        pltpu.VMEM((GCH,), jnp.int32),
        pltpu.VMEM((GCH, C), jnp.float32),
        pltpu.VMEM((GCH, C), jnp.float32),
        pltpu.VMEM((2, C, GCH), jnp.float32),
        pltpu.VMEM((2, C, GCH), jnp.float32),
        pltpu.VMEM((GTL,), jnp.int32),
        pltpu.VMEM((GTL,), jnp.int32),
        pltpu.VMEM((GTL, C), jnp.float32),
        pltpu.VMEM((GTL, C), jnp.float32),
        pltpu.VMEM((C, GTL), jnp.float32),
        pltpu.VMEM((C, GTL), jnp.float32),
        pltpu.SemaphoreType.DMA,
        pltpu.SemaphoreType.DMA((2,)),
    ],
    compiler_params=_SC_PARAMS,
    name="sc_gather_hn",
)


def _untranspose_cols(tb, rows, ngrp):
    def grp(g, carry):
        rb = g * 16
        ridx = rb + lax.iota(jnp.int32, 16)
        for w in range(16):
            widx = jnp.full((16,), w, jnp.int32)
            v = plsc.load_gather(tb, [widx, ridx])
            plsc.store_scatter(rows, [ridx, widx], v)
        return carry
    lax.fori_loop(0, ngrp, grp, 0)


def _scatter_body(valst_hbm, dsti_hbm, out_hbm, shared, zbuf,
                  idx_b0, tld_b0, val_b0, idx_b1, tld_b1, val_b1,
                  idx_t, tld_t, val_t, sem_s):
    cid = lax.axis_index("c")
    sid = lax.axis_index("s")
    wid = sid * NC + cid

    def zrow(i, carry):
        zbuf[i, :] = jnp.zeros((C,), jnp.float32)
        return carry

    lax.fori_loop(0, RPS, zrow, 0)
    pltpu.sync_copy(zbuf, shared.at[pl.ds(sid * RPS, RPS)])
    plsc.subcore_barrier()

    base = wid * EPW
    slot = [(idx_b0, tld_b0, val_b0), (idx_b1, tld_b1, val_b1)]

    def half(j, p):
        ib, tb, vb = slot[p]

        @pl.when(j >= 2)
        def _():
            pltpu.make_async_copy(vb, shared.at[ib], sem_s.at[p]).wait()

        off = base + j * SCH
        pltpu.sync_copy(dsti_hbm.at[pl.ds(off, SCH)], ib)
        pltpu.sync_copy(valst_hbm.at[:, pl.ds(off, SCH)], tb)
        _untranspose_cols(tb, vb, SCH // 16)
        pltpu.async_copy(vb, shared.at[ib], sem_s.at[p], add=True)

    def pair(t, carry):
        half(2 * t, 0)
        half(2 * t + 1, 1)
        return carry

    lax.fori_loop(0, SNF // 2, pair, 0)
    half(SNF - 1, 0)

    for k in (1, 0):
        ib, _, vb = slot[k]
        pltpu.make_async_copy(vb, shared.at[ib], sem_s.at[k]).wait()

    off = base + SNF * SCH
    pltpu.sync_copy(dsti_hbm.at[pl.ds(off, STL)], idx_t)
    pltpu.sync_copy(valst_hbm.at[:, pl.ds(off, STL)], tld_t)
    ridx = lax.iota(jnp.int32, 16)
    msk = ridx < STL
    for w in range(16):
        widx = jnp.full((16,), w, jnp.int32)
        v = plsc.load_gather(tld_t, [widx, ridx], mask=msk)
        plsc.store_scatter(val_t, [ridx, widx], v, mask=msk)
    pltpu.sync_copy(val_t.at[pl.ds(0, STL)], shared.at[idx_t], add=True)

    plsc.subcore_barrier()
    pltpu.sync_copy(shared.at[pl.ds(sid * RPS, RPS)],
                    out_hbm.at[pl.ds(cid * NPAD + sid * RPS, RPS)])


_sc_scatter = pl.kernel(
    _scatter_body,
    out_type=jax.ShapeDtypeStruct((NC * NPAD, C), jnp.float32),
    mesh=_SC_MESH,
    scratch_types=[
        pltpu.VMEM_SHARED((NPAD, C), jnp.float32),
        pltpu.VMEM((RPS, C), jnp.float32),
        pltpu.VMEM((SCH,), jnp.int32),
        pltpu.VMEM((C, SCH), jnp.float32),
        pltpu.VMEM((SCH, C), jnp.float32),
        pltpu.VMEM((SCH,), jnp.int32),
        pltpu.VMEM((C, SCH), jnp.float32),
        pltpu.VMEM((SCH, C), jnp.float32),
        pltpu.VMEM((STL,), jnp.int32),
        pltpu.VMEM((C, STL), jnp.float32),
        pltpu.VMEM((16, C), jnp.float32),
        pltpu.SemaphoreType.DMA((2,)),
    ],
    compiler_params=_SC_PARAMS,
    name="sc_scatter_nodeftr",
)


BT = 3200
_SQRT2 = math.sqrt(2.0)
_INV_SQRT48 = 1.0 / math.sqrt(48.0)
_BF = jnp.bfloat16


def _rep16(x):
    b = x.shape[1]
    return jnp.broadcast_to(x[:, None, :], (16, 16, b)).reshape(256, b)


def _sum16(x):
    b = x.shape[1]
    return x.reshape(16, 16, b).sum(axis=0)


def _edge_body(het_ref, hst_ref, hdt_ref, embt_ref, nrmt_ref,
               w12t_ref, m1t_ref, m2t_ref, henewt_ref, hesct_ref):
    het_b = het_ref[...]
    embt_b = embt_ref[...].astype(_BF)
    nrmt_b = nrmt_ref[...]

    h12 = jnp.maximum(jax.lax.dot(w12t_ref[...], embt_b,
                                  preferred_element_type=jnp.float32),
                      0.0) * _SQRT2
    h1t = h12[0:16, :]
    h2t = h12[16:32, :]

    featt = jnp.concatenate([het_b, hst_ref[...], hdt_ref[...]],
                            axis=0).astype(_BF)
    g1t = jax.lax.dot(m1t_ref[...], featt,
                      preferred_element_type=jnp.float32)
    tmpt = jnp.maximum(_sum16(_rep16(h1t) * g1t) * _INV_SQRT48, 0.0)

    g2t = jax.lax.dot(m2t_ref[...], tmpt.astype(_BF),
                      preferred_element_type=jnp.float32)
    he_newt = het_b + _sum16(_rep16(h2t) * g2t) * 0.25

    henewt_ref[...] = he_newt
    hesct_ref[...] = he_newt * nrmt_b


def _edge_mlp(het, hst, hdt, embt, nrmt, w12t, m1t, m2t):
    grid = (N_EDGES // BT,)
    col = lambda r: pl.BlockSpec((r, BT), lambda i: (0, i))
    full = lambda r, c: pl.BlockSpec((r, c), lambda i: (0, 0))
    return pl.pallas_call(
        _edge_body,
        grid=grid,
        in_specs=[col(C), col(C), col(C), col(10), col(1),
                  full(32, 10), full(256, 48), full(256, C)],
        out_specs=[col(C), col(C)],
        out_shape=(jax.ShapeDtypeStruct((C, N_EDGES), jnp.float32),
                   jax.ShapeDtypeStruct((C, N_EDGES), jnp.float32)),
        name="tc_edge_mlp",
    )(het, hst, hdt, embt, nrmt, w12t, m1t, m2t)


def _node_body(hnt_ref, nf0t_ref, nf1t_ref, l1t_ref, l2t_ref, out_ref):
    hnt_b = hnt_ref[...]
    nft = nf0t_ref[...] + nf1t_ref[...]
    cat = jnp.concatenate([hnt_b, nft], axis=0)
    h = jnp.maximum(jax.lax.dot(l1t_ref[...], cat,
                                preferred_element_type=jnp.float32),
                    0.0) * _SQRT2
    out_ref[...] = hnt_b + jax.lax.dot(l2t_ref[...], h,
                                       preferred_element_type=jnp.float32)


def _node_update(hnt, nf0t, nf1t, l1t, l2t):
    return pl.pallas_call(
        _node_body,
        out_shape=jax.ShapeDtypeStruct((C, N_NODES), jnp.float32),
        name="tc_node_update",
    )(hnt, nf0t, nf1t, l1t, l2t)


def kernel(hn, he, edge_vec, emb, norm, fc1_w1, fc1_w2, fc2_w1, fc2_w2,
           lin1_w, lin2_w, edge_index):
    del edge_vec
    src = edge_index[0]
    dst = edge_index[1]

    w12t = (jnp.concatenate([fc1_w1, fc2_w1], axis=1).T
            * (1.0 / math.sqrt(10.0))).astype(_BF)
    m1t = ((fc1_w2 * 0.25).reshape(16, 48, 16).transpose(1, 0, 2)
           .reshape(48, 256).T).astype(_BF)
    m2t = ((fc2_w2 * 0.25).reshape(16, 16, 16).transpose(1, 0, 2)
           .reshape(16, 256).T).astype(_BF)
    l1t = lin1_w.T * (1.0 / math.sqrt(32.0))
    l2t = lin2_w.T * 0.25

    hst, hdt = _sc_gather(hn, src, dst)
    he_newt, he_scaledt = _edge_mlp(he.T, hst, hdt, emb.T,
                                    norm.reshape(1, N_EDGES),
                                    w12t, m1t, m2t)
    nf_parts = _sc_scatter(he_scaledt, dst)
    nf = nf_parts.reshape(NC, NPAD, C)
    hn_newt = _node_update(hn.T, nf[0, :N_NODES].T, nf[1, :N_NODES].T,
                           l1t, l2t)
    return hn_newt.T, he_newt.T
